# baseline (speedup 1.0000x reference)
import functools

import jax
import jax.numpy as jnp
from jax import lax
from jax.experimental import pallas as pl
from jax.experimental.pallas import tpu as pltpu
from jax.experimental.pallas import tpu_sc as plsc

_NUM_CORES = 2
_NUM_SUBCORES = 16
_NUM_WORKERS = _NUM_CORES * _NUM_SUBCORES
_LANES = 16

_D = 10
_TAB_ROWS = 16
_TAB_COLS = 1024
_UB = 256
_UL = 8


def _fuse_table_body(table_ref, w_ref, b_ref, out_ref):
    fused_t = (
        lax.dot_general(
            w_ref[...], table_ref[...],
            dimension_numbers=(((1,), (1,)), ((), ())),
            preferred_element_type=jnp.float32,
        )
        + b_ref[...]
    )
    out_ref[...] = jnp.pad(
        fused_t,
        ((0, _TAB_ROWS - fused_t.shape[0]),
         (0, _TAB_COLS - fused_t.shape[1])))


def _fuse_table(table, W, b_col):
    return pl.pallas_call(
        _fuse_table_body,
        out_shape=jax.ShapeDtypeStruct((_TAB_ROWS, _TAB_COLS), jnp.float32),
    )(table, W, b_col)


def _make_gather(batch, hist):
    bcol = batch // _NUM_WORKERS
    assert batch % (_NUM_WORKERS * _UB) == 0
    assert bcol % _UB == 0 and hist % _UL == 0
    halves = bcol // _UB
    lgroups = hist // _UL
    n_units = halves * lgroups
    assert n_units % 2 == 0

    mesh = plsc.VectorSubcoreMesh(
        core_axis_name="c", subcore_axis_name="s",
        num_cores=_NUM_CORES, num_subcores=_NUM_SUBCORES,
    )

    @functools.partial(
        pl.kernel,
        mesh=mesh,
        compiler_params=pltpu.CompilerParams(needs_layout_passes=False),
        out_type=jax.ShapeDtypeStruct((_D, hist, batch), jnp.float32),
        scratch_types=[
            pltpu.VMEM((_TAB_ROWS, _TAB_COLS), jnp.float32),
            pltpu.VMEM((_UL, _UB), jnp.int32),
            pltpu.VMEM((_UL, _UB), jnp.int32),
            pltpu.VMEM((_D, _UL, _UB), jnp.float32),
            pltpu.VMEM((_D, _UL, _UB), jnp.float32),
            pltpu.SemaphoreType.DMA,
            pltpu.SemaphoreType.DMA,
            pltpu.SemaphoreType.DMA,
            pltpu.SemaphoreType.DMA,
        ],
    )
    def gather_kernel(idx_hbm, tab_hbm, out_hbm,
                      tab_v, idx_a, idx_b, out_a, out_b,
                      sin_a, sin_b, sout_a, sout_b):
        wid = lax.axis_index("s") * _NUM_CORES + lax.axis_index("c")
        wb0 = wid * bcol
        pltpu.sync_copy(tab_hbm, tab_v)
        dvs = [jnp.full((_LANES,), d, jnp.int32) for d in range(_D)]

        def unit_slices(u):
            lg = u // halves
            half = u % halves
            b0 = wb0 + half * _UB
            l0 = lg * _UL
            return (idx_hbm.at[pl.ds(l0, _UL), pl.ds(b0, _UB)],
                    out_hbm.at[:, pl.ds(l0, _UL), pl.ds(b0, _UB)])

        i0, _ = unit_slices(0)
        pltpu.async_copy(i0, idx_a, sin_a)
        i1, _ = unit_slices(1)
        pltpu.async_copy(i1, idx_b, sin_b)

        def pair_body(g, carry):
            for j, idx_v, out_v, sin, sout in (
                    (0, idx_a, out_a, sin_a, sout_a),
                    (1, idx_b, out_b, sin_b, sout_b)):
                u = 2 * g + j
                isl, osl = unit_slices(u)
                pltpu.make_async_copy(isl, idx_v, sin).wait()
                @pl.when(g >= 1)
                def _():
                    _, osl_prev = unit_slices(u - 2)
                    pltpu.make_async_copy(out_v, osl_prev, sout).wait()

                @plsc.parallel_loop(0, _UL * (_UB // _LANES), 1, unroll=4)
                def l_body(i):
                    l = i >> 4
                    boff = (i & 15) * _LANES
                    idx16 = idx_v[l, pl.ds(boff, _LANES)]
                    vals = [plsc.load_gather(tab_v, [dvs[d], idx16])
                            for d in range(_D)]
                    for d in range(_D):
                        out_v[d, l, pl.ds(boff, _LANES)] = vals[d]
                pltpu.async_copy(out_v, osl, sout)

                @pl.when(g <= n_units // 2 - 2)
                def _():
                    isl_next, _ = unit_slices(u + 2)
                    pltpu.async_copy(isl_next, idx_v, sin)
            return carry

        lax.fori_loop(0, n_units // 2, pair_body, 0)
        _, osl_a = unit_slices(n_units - 2)
        pltpu.make_async_copy(out_a, osl_a, sout_a).wait()
        _, osl_b = unit_slices(n_units - 1)
        pltpu.make_async_copy(out_b, osl_b, sout_b).wait()

    return gather_kernel


def kernel(x, table, W, b):
    batch, hist = x.shape
    fused_t = _fuse_table(table, W, b.reshape(-1, 1))
    gather = _make_gather(batch, hist)
    out_t = gather(x.T, fused_t)
    return jnp.transpose(out_t, (2, 1, 0))


# speedup vs baseline: 1.3464x; 1.3464x over previous
import functools

import jax
import jax.numpy as jnp
from jax import lax
from jax.experimental import pallas as pl
from jax.experimental.pallas import tpu as pltpu
from jax.experimental.pallas import tpu_sc as plsc

_NUM_CORES = 2
_NUM_SUBCORES = 16
_NUM_WORKERS = _NUM_CORES * _NUM_SUBCORES
_LANES = 16

_D = 10
_TAB_ROWS = 16
_TAB_COLS = 1024
_UB = 256
_UL = 8


def _fuse_table_body(table_ref, w_ref, b_ref, out_ref):
    fused_t = (
        lax.dot_general(
            w_ref[...], table_ref[...],
            dimension_numbers=(((1,), (1,)), ((), ())),
            preferred_element_type=jnp.float32,
        )
        + b_ref[...]
    )
    out_ref[...] = jnp.pad(
        fused_t,
        ((0, _TAB_ROWS - fused_t.shape[0]),
         (0, _TAB_COLS - fused_t.shape[1])))


def _fuse_table(table, W, b_col):
    return pl.pallas_call(
        _fuse_table_body,
        out_shape=jax.ShapeDtypeStruct((_TAB_ROWS, _TAB_COLS), jnp.float32),
    )(table, W, b_col)


def _make_gather(batch, hist):
    bcol = batch // _NUM_WORKERS
    assert batch % (_NUM_WORKERS * _UB) == 0
    assert bcol % _UB == 0 and hist % _UL == 0
    halves = bcol // _UB
    lgroups = hist // _UL
    n_units = halves * lgroups
    assert n_units % 2 == 0

    mesh = plsc.VectorSubcoreMesh(
        core_axis_name="c", subcore_axis_name="s",
        num_cores=_NUM_CORES, num_subcores=_NUM_SUBCORES,
    )

    @functools.partial(
        pl.kernel,
        mesh=mesh,
        compiler_params=pltpu.CompilerParams(needs_layout_passes=False),
        out_type=jax.ShapeDtypeStruct((_D, hist, batch), jnp.float32),
        scratch_types=[
            pltpu.VMEM((_TAB_ROWS, _TAB_COLS), jnp.float32),
            pltpu.VMEM((_UL * _UB,), jnp.int32),
            pltpu.VMEM((_UL * _UB,), jnp.int32),
            pltpu.VMEM((_D, _UL * _UB), jnp.float32),
            pltpu.VMEM((_D, _UL * _UB), jnp.float32),
            pltpu.SemaphoreType.DMA,
            pltpu.SemaphoreType.DMA,
            pltpu.SemaphoreType.DMA,
            pltpu.SemaphoreType.DMA,
        ],
    )
    def gather_kernel(idx_hbm, tab_hbm, out_hbm,
                      tab_v, idx_a, idx_b, out_a, out_b,
                      sin_a, sin_b, sout_a, sout_b):
        wid = lax.axis_index("s") * _NUM_CORES + lax.axis_index("c")
        wb0 = wid * bcol
        pltpu.sync_copy(tab_hbm, tab_v)
        dvs = [jnp.full((_LANES,), d, jnp.int32) for d in range(_D)]

        def in_copies(u, idx_v):
            b0 = wb0 + (u % halves) * _UB
            l0 = (u // halves) * _UL
            return [(idx_hbm.at[l0 + r, pl.ds(b0, _UB)],
                     idx_v.at[pl.ds(r * _UB, _UB)])
                    for r in range(_UL)]

        def out_copies(u, out_v):
            b0 = wb0 + (u % halves) * _UB
            l0 = (u // halves) * _UL
            return [(out_v.at[:, pl.ds(r * _UB, _UB)],
                     out_hbm.at[:, l0 + r, pl.ds(b0, _UB)])
                    for r in range(_UL)]

        for s, d in in_copies(0, idx_a):
            pltpu.async_copy(s, d, sin_a)
        for s, d in in_copies(1, idx_b):
            pltpu.async_copy(s, d, sin_b)

        def pair_body(g, carry):
            for j, idx_v, out_v, sin, sout in (
                    (0, idx_a, out_a, sin_a, sout_a),
                    (1, idx_b, out_b, sin_b, sout_b)):
                u = 2 * g + j
                for s, d in in_copies(u, idx_v):
                    pltpu.make_async_copy(s, d, sin).wait()
                @pl.when(g >= 1)
                def _():
                    for s, d in out_copies(u - 2, out_v):
                        pltpu.make_async_copy(s, d, sout).wait()

                @plsc.parallel_loop(0, _UL * (_UB // _LANES), 1, unroll=2)
                def l_body(i):
                    boff = i * _LANES
                    idx16 = idx_v[pl.ds(boff, _LANES)]
                    vals = [plsc.load_gather(tab_v, [dvs[d], idx16])
                            for d in range(_D)]
                    for d in range(_D):
                        out_v[d, pl.ds(boff, _LANES)] = vals[d]

                for s, d in out_copies(u, out_v):
                    pltpu.async_copy(s, d, sout)

                @pl.when(g <= n_units // 2 - 2)
                def _():
                    for s, d in in_copies(u + 2, idx_v):
                        pltpu.async_copy(s, d, sin)
            return carry

        lax.fori_loop(0, n_units // 2, pair_body, 0)
        for s, d in out_copies(n_units - 2, out_a):
            pltpu.make_async_copy(s, d, sout_a).wait()
        for s, d in out_copies(n_units - 1, out_b):
            pltpu.make_async_copy(s, d, sout_b).wait()

    return gather_kernel


def kernel(x, table, W, b):
    batch, hist = x.shape
    fused_t = _fuse_table(table, W, b.reshape(-1, 1))
    gather = _make_gather(batch, hist)
    out_t = gather(x.T, fused_t)
    return jnp.transpose(out_t, (2, 1, 0))
